# baseline (speedup 1.0000x reference)
import functools

import jax
import jax.numpy as jnp
from jax import lax
from jax.experimental import pallas as pl
from jax.experimental.pallas import tpu as pltpu
from jax.experimental.pallas import tpu_sc as plsc

_NC = 2
_NS = 16
_NW = _NC * _NS
_L = 16
_TILE = 128


def _matvec_body(z_ref, w_ref, b_ref, y_ref):
    prod = jax.lax.dot_general(
        w_ref[...],
        z_ref[...],
        dimension_numbers=(((1,), (1,)), ((), ())),
        preferred_element_type=jnp.float32,
    )
    y_ref[...] = jnp.exp(prod + b_ref[0, 0] * 0.5)


def _edge_body(y_hbm, ei_hbm, out_hbm, y_v, idx_v, out_v):
    cw = idx_v.shape[1]
    e = ei_hbm.shape[1]
    ntiles = e // _TILE
    wid = lax.axis_index("s") * _NC + lax.axis_index("c")
    base = (wid * ntiles) // _NW * _TILE
    pltpu.sync_copy(y_hbm.at[0], y_v)
    pltpu.sync_copy(ei_hbm.at[:, pl.ds(base, cw)], idx_v)

    @plsc.parallel_loop(0, cw, _L, unroll=16)
    def step(off):
        sv = plsc.load_gather(y_v, [idx_v[0, pl.ds(off, _L)]])
        dv = plsc.load_gather(y_v, [idx_v[1, pl.ds(off, _L)]])
        p = sv * dv
        out_v[pl.ds(off, _L)] = 1.0 - 1.0 / (1.0 + p)

    pltpu.sync_copy(out_v, out_hbm.at[0, pl.ds(base, cw)])


def kernel(z, edge_index, W, b):
    n, d = z.shape
    e = edge_index.shape[1]
    ntiles = e // _TILE
    cw = (ntiles + _NW - 1) // _NW * _TILE

    blk = 1280
    nblk = (n + blk - 1) // blk
    y = pl.pallas_call(
        _matvec_body,
        grid=(nblk,),
        in_specs=[
            pl.BlockSpec((blk, d), lambda i: (i, 0)),
            pl.BlockSpec((1, d), lambda i: (0, 0)),
            pl.BlockSpec((1, 1), lambda i: (0, 0)),
        ],
        out_specs=pl.BlockSpec((1, blk), lambda i: (0, i)),
        out_shape=jax.ShapeDtypeStruct((1, n), jnp.float32),
    )(z, W, b.reshape(1, 1))

    ei = edge_index.astype(jnp.int32)

    edge_kernel = pl.kernel(
        _edge_body,
        out_type=jax.ShapeDtypeStruct((1, e), jnp.float32),
        mesh=plsc.VectorSubcoreMesh(
            core_axis_name="c", subcore_axis_name="s"
        ),
        compiler_params=pltpu.CompilerParams(
            needs_layout_passes=False,
            skip_device_barrier=True,
            disable_bounds_checks=True,
            disable_semaphore_checks=True,
        ),
        scratch_types=[
            pltpu.VMEM((n,), jnp.float32),
            pltpu.VMEM((2, cw), jnp.int32),
            pltpu.VMEM((cw,), jnp.float32),
        ],
    )
    out = edge_kernel(y, ei)
    return out.reshape(e, 1)


# speedup vs baseline: 1.1000x; 1.1000x over previous
import functools

import jax
import jax.numpy as jnp
from jax import lax
from jax.experimental import pallas as pl
from jax.experimental.pallas import tpu as pltpu
from jax.experimental.pallas import tpu_sc as plsc

_NC = 2
_NS = 16
_NW = _NC * _NS
_L = 16
_TILE = 128


def _matvec_body(z_ref, w_ref, b_ref, y_ref):
    prod = jax.lax.dot_general(
        w_ref[...],
        z_ref[...],
        dimension_numbers=(((1,), (1,)), ((), ())),
        preferred_element_type=jnp.float32,
    )
    y_ref[...] = jnp.exp(prod + b_ref[0, 0] * 0.5)


def _edge_body(y_hbm, ei_hbm, out_hbm, y_v, idx_v, out_v):
    cw = idx_v.shape[1]
    e = ei_hbm.shape[1]
    ntiles = e // _TILE
    wid = lax.axis_index("s") * _NC + lax.axis_index("c")
    base = (wid * ntiles) // _NW * _TILE
    pltpu.sync_copy(y_hbm.at[0], y_v)
    pltpu.sync_copy(ei_hbm.at[:, pl.ds(base, cw)], idx_v)

    @plsc.parallel_loop(0, cw, _L, unroll=16)
    def step(off):
        sv = plsc.load_gather(y_v, [idx_v[0, pl.ds(off, _L)]])
        dv = plsc.load_gather(y_v, [idx_v[1, pl.ds(off, _L)]])
        p = sv * dv
        out_v[pl.ds(off, _L)] = 1.0 - 1.0 / (1.0 + p)

    pltpu.sync_copy(out_v, out_hbm.at[0, pl.ds(base, cw)])


def kernel(z, edge_index, W, b):
    n, d = z.shape
    e = edge_index.shape[1]
    ntiles = e // _TILE
    cw = (ntiles + _NW - 1) // _NW * _TILE

    y = pl.pallas_call(
        _matvec_body,
        out_shape=jax.ShapeDtypeStruct((1, n), jnp.float32),
    )(z, W, b.reshape(1, 1))

    ei = edge_index.astype(jnp.int32)

    edge_kernel = pl.kernel(
        _edge_body,
        out_type=jax.ShapeDtypeStruct((1, e), jnp.float32),
        mesh=plsc.VectorSubcoreMesh(
            core_axis_name="c", subcore_axis_name="s"
        ),
        compiler_params=pltpu.CompilerParams(
            needs_layout_passes=False,
            skip_device_barrier=True,
            disable_bounds_checks=True,
            disable_semaphore_checks=True,
        ),
        scratch_types=[
            pltpu.VMEM((n,), jnp.float32),
            pltpu.VMEM((2, cw), jnp.int32),
            pltpu.VMEM((cw,), jnp.float32),
        ],
    )
    out = edge_kernel(y, ei)
    return out.reshape(e, 1)


# speedup vs baseline: 1.1382x; 1.0348x over previous
import functools

import jax
import jax.numpy as jnp
from jax import lax
from jax.experimental import pallas as pl
from jax.experimental.pallas import tpu as pltpu
from jax.experimental.pallas import tpu_sc as plsc

_NC = 2
_NS = 16
_NW = _NC * _NS
_L = 16
_TILE = 128


def _matvec_body(z_ref, w_ref, b_ref, y_ref):
    prod = jax.lax.dot_general(
        w_ref[...],
        z_ref[...],
        dimension_numbers=(((1,), (1,)), ((), ())),
        preferred_element_type=jnp.float32,
    )
    y_ref[...] = jnp.exp(prod + b_ref[0, 0] * 0.5)


def _edge_body(y_hbm, ei_hbm, out_hbm, y_v, idx_v, out_v, sem_y, sem_i0,
               sem_i1, sem_o):
    cw = idx_v.shape[1]
    c0 = (cw // (2 * _TILE) + 1) * _TILE
    c1 = cw - c0
    e = ei_hbm.shape[1]
    ntiles = e // _TILE
    wid = lax.axis_index("s") * _NC + lax.axis_index("c")
    base = (wid * ntiles) // _NW * _TILE

    cp_y = pltpu.make_async_copy(y_hbm.at[0], y_v, sem_y)
    cp_i0 = pltpu.make_async_copy(
        ei_hbm.at[:, pl.ds(base, c0)], idx_v.at[:, pl.ds(0, c0)], sem_i0
    )
    cp_i1 = pltpu.make_async_copy(
        ei_hbm.at[:, pl.ds(base + c0, c1)],
        idx_v.at[:, pl.ds(c0, c1)],
        sem_i1,
    )
    cp_y.start()
    cp_i0.start()
    cp_i1.start()
    cp_y.wait()
    cp_i0.wait()

    @plsc.parallel_loop(0, c0, _L, unroll=16)
    def step0(off):
        sv = plsc.load_gather(y_v, [idx_v[0, pl.ds(off, _L)]])
        dv = plsc.load_gather(y_v, [idx_v[1, pl.ds(off, _L)]])
        p = sv * dv
        out_v[pl.ds(off, _L)] = 1.0 - 1.0 / (1.0 + p)

    cp_o0 = pltpu.make_async_copy(
        out_v.at[pl.ds(0, c0)], out_hbm.at[0, pl.ds(base, c0)], sem_o
    )
    cp_o0.start()
    cp_i1.wait()

    @plsc.parallel_loop(c0, cw, _L, unroll=16)
    def step1(off):
        sv = plsc.load_gather(y_v, [idx_v[0, pl.ds(off, _L)]])
        dv = plsc.load_gather(y_v, [idx_v[1, pl.ds(off, _L)]])
        p = sv * dv
        out_v[pl.ds(off, _L)] = 1.0 - 1.0 / (1.0 + p)

    cp_o1 = pltpu.make_async_copy(
        out_v.at[pl.ds(c0, c1)], out_hbm.at[0, pl.ds(base + c0, c1)], sem_o
    )
    cp_o1.start()
    cp_o0.wait()
    cp_o1.wait()


def kernel(z, edge_index, W, b):
    n, d = z.shape
    e = edge_index.shape[1]
    ntiles = e // _TILE
    cw = (ntiles + _NW - 1) // _NW * _TILE

    y = pl.pallas_call(
        _matvec_body,
        out_shape=jax.ShapeDtypeStruct((1, n), jnp.float32),
    )(z, W, b.reshape(1, 1))

    ei = edge_index.astype(jnp.int32)

    edge_kernel = pl.kernel(
        _edge_body,
        out_type=jax.ShapeDtypeStruct((1, e), jnp.float32),
        mesh=plsc.VectorSubcoreMesh(
            core_axis_name="c", subcore_axis_name="s"
        ),
        compiler_params=pltpu.CompilerParams(
            needs_layout_passes=False,
            skip_device_barrier=True,
            disable_bounds_checks=True,
            disable_semaphore_checks=True,
        ),
        scratch_types=[
            pltpu.VMEM((n,), jnp.float32),
            pltpu.VMEM((2, cw), jnp.int32),
            pltpu.VMEM((cw,), jnp.float32),
            pltpu.SemaphoreType.DMA,
            pltpu.SemaphoreType.DMA,
            pltpu.SemaphoreType.DMA,
            pltpu.SemaphoreType.DMA,
        ],
    )
    out = edge_kernel(y, ei)
    return out.reshape(e, 1)
